# baseline (speedup 1.0000x reference)
import functools

import jax
import jax.numpy as jnp
from jax import lax
from jax.experimental import pallas as pl
from jax.experimental.pallas import tpu as pltpu
from jax.experimental.pallas import tpu_sc as plsc

T, H, I2, E, K = 32, 768, 1536, 64, 2
I = I2 // 2
S = K * T
NBUF = 3
L = 16
NC = 2


def _sc_router_body(logits_hbm, m_out, im_out, s_out, is_out,
                    lv, mv, imv, sv, isv):
    wid = lax.axis_index("s") * NC + lax.axis_index("c")
    pltpu.sync_copy(logits_hbm.at[wid], lv)

    iota = lax.iota(jnp.int32, L)
    m = lv[pl.ds(0, L)]
    im = iota
    s = jnp.full((L,), -jnp.inf, jnp.float32)
    is_ = jnp.zeros((L,), jnp.int32)
    for j in range(1, E // L):
        v = lv[pl.ds(L * j, L)]
        iv = L * j + iota
        ge = m >= v
        loser = jnp.where(ge, v, m)
        il = jnp.where(ge, iv, im)
        m = jnp.where(ge, m, v)
        im = jnp.where(ge, im, iv)
        ge2 = (loser > s) | ((loser == s) & (il < is_))
        s = jnp.where(ge2, loser, s)
        is_ = jnp.where(ge2, il, is_)

    mv[...] = m
    imv[...] = im
    sv[...] = s
    isv[...] = is_
    pltpu.sync_copy(mv, m_out.at[wid])
    pltpu.sync_copy(imv, im_out.at[wid])
    pltpu.sync_copy(sv, s_out.at[wid])
    pltpu.sync_copy(isv, is_out.at[wid])


_sc_router = functools.partial(
    pl.kernel,
    out_type=(
        jax.ShapeDtypeStruct((T, L), jnp.float32),
        jax.ShapeDtypeStruct((T, L), jnp.int32),
        jax.ShapeDtypeStruct((T, L), jnp.float32),
        jax.ShapeDtypeStruct((T, L), jnp.int32),
    ),
    mesh=plsc.VectorSubcoreMesh(core_axis_name="c", subcore_axis_name="s"),
    scratch_types=[
        pltpu.VMEM((E,), jnp.float32),
        pltpu.VMEM((L,), jnp.float32),
        pltpu.VMEM((L,), jnp.int32),
        pltpu.VMEM((L,), jnp.float32),
        pltpu.VMEM((L,), jnp.int32),
    ],
)(_sc_router_body)


def _row_of(col, n):
    i = jax.lax.broadcasted_iota(jnp.int32, (n, n), 0)
    j = jax.lax.broadcasted_iota(jnp.int32, (n, n), 1)
    b = jnp.broadcast_to(col, (n, n))
    return jnp.sum(jnp.where(i == j, b, jnp.zeros_like(b)), axis=0,
                   keepdims=True)


def _moe_body(m_ref, im_ref, s_ref, is_ref, x_ref, gup_ref, dnp_ref, out_ref,
              wv, uv, nv, us, ns, gbuf, dbuf, ssem, nsem, gsem, dsem):
    m = m_ref[...]
    im = im_ref[...]
    s = s_ref[...]
    is_ = is_ref[...]
    m1v = jnp.max(m, axis=1, keepdims=True)
    i1 = jnp.min(jnp.where(m == m1v, im, E), axis=1, keepdims=True)
    chosen = im == i1
    cand = jnp.where(chosen, s, m)
    icand = jnp.where(chosen, is_, im)
    m2v = jnp.max(cand, axis=1, keepdims=True)
    i2 = jnp.min(jnp.where(cand == m2v, icand, E), axis=1, keepdims=True)
    e2 = jnp.exp(m2v - m1v)
    w1 = 1.0 / (1.0 + e2)
    w2 = 1.0 - w1

    e_col = jnp.concatenate([i1, i2], axis=0)
    e_row = _row_of(e_col, S)
    s_col = jax.lax.broadcasted_iota(jnp.int32, (S, S), 0)
    s_row = jax.lax.broadcasted_iota(jnp.int32, (S, S), 1)
    e_colb = jnp.broadcast_to(e_col, (S, S))
    e_rowb = jnp.broadcast_to(e_row, (S, S))
    same = e_rowb == e_colb
    dup_cnt = jnp.sum((same & (s_row < s_col)).astype(jnp.int32), axis=1,
                      keepdims=True)
    first = (dup_cnt == 0).astype(jnp.int32)
    firstb = jnp.broadcast_to(_row_of(first, S), (S, S))
    d = jnp.sum(((firstb == 1) & (e_rowb < e_colb)).astype(jnp.int32),
                axis=1, keepdims=True)
    j_row = jax.lax.broadcasted_iota(jnp.int32, (S, S), 1)
    put = (jnp.broadcast_to(d, (S, S)) == j_row) & (
        jnp.broadcast_to(first, (S, S)) == 1)
    uv[...] = jnp.sum(jnp.where(put, e_colb, jnp.zeros_like(e_colb)),
                      axis=0, keepdims=True)
    nv[...] = jnp.sum(first, keepdims=True)

    pltpu.make_async_copy(uv, us, ssem).start()
    pltpu.make_async_copy(nv, ns, nsem).start()
    pltpu.make_async_copy(uv, us, ssem).wait()
    pltpu.make_async_copy(nv, ns, nsem).wait()

    nu = ns[0, 0]

    def start_copy(u, slot):
        e = us[0, u]
        pltpu.make_async_copy(gup_ref.at[pl.ds(e, 1)],
                              gbuf.at[pl.ds(slot, 1)], gsem.at[slot]).start()
        pltpu.make_async_copy(dnp_ref.at[pl.ds(e, 1)],
                              dbuf.at[pl.ds(slot, 1)], dsem.at[slot]).start()

    for b in range(NBUF - 1):
        @pl.when(b < nu)
        def _pro():
            start_copy(b, b)

    i1r = jnp.broadcast_to(_row_of(i1, T), (E, T))
    i2r = jnp.broadcast_to(_row_of(i2, T), (E, T))
    w1r = jnp.broadcast_to(_row_of(w1, T), (E, T))
    w2r = jnp.broadcast_to(_row_of(w2, T), (E, T))
    e_iota = jax.lax.broadcasted_iota(jnp.int32, (E, T), 0)
    zero = jnp.zeros((E, T), jnp.float32)
    wv[...] = (jnp.where(e_iota == i1r, w1r, zero)
               + jnp.where(e_iota == i2r, w2r, zero))

    out_ref[...] = jnp.zeros_like(out_ref)

    def body(u, _):
        nxt = u + NBUF - 1

        @pl.when(nxt < nu)
        def _issue():
            start_copy(nxt, jax.lax.rem(nxt, NBUF))

        slot = jax.lax.rem(u, NBUF)
        e = us[0, u]
        pltpu.make_async_copy(gup_ref.at[pl.ds(e, 1)],
                              gbuf.at[pl.ds(slot, 1)], gsem.at[slot]).wait()
        pltpu.make_async_copy(dnp_ref.at[pl.ds(e, 1)],
                              dbuf.at[pl.ds(slot, 1)], dsem.at[slot]).wait()

        g = gbuf[slot]
        gu = jax.lax.dot_general(
            x_ref[...], g, (((1,), (1,)), ((), ())),
            preferred_element_type=jnp.float32)
        gate = gu[:, :I]
        up = gu[:, I:]
        act = gate * jax.nn.sigmoid(gate) * up
        dn = dbuf[slot]
        eo = jax.lax.dot_general(
            act, dn, (((1,), (1,)), ((), ())),
            preferred_element_type=jnp.float32)
        wrow = wv[pl.ds(e, 1), :]
        wcol = _col_of(wrow, T)
        out_ref[...] += wcol * eo
        return 0

    jax.lax.fori_loop(0, nu, body, 0)


def _col_of(row, n):
    i = jax.lax.broadcasted_iota(jnp.int32, (n, n), 0)
    j = jax.lax.broadcasted_iota(jnp.int32, (n, n), 1)
    b = jnp.broadcast_to(row, (n, n))
    return jnp.sum(jnp.where(i == j, b, jnp.zeros_like(b)), axis=1,
                   keepdims=True)


@jax.jit
def _fused_moe(x, router_logits, gate_up_proj, down_proj):
    mm, im, ss, iss = _sc_router(router_logits)
    return pl.pallas_call(
        _moe_body,
        in_specs=[
            pl.BlockSpec(memory_space=pltpu.VMEM),
            pl.BlockSpec(memory_space=pltpu.VMEM),
            pl.BlockSpec(memory_space=pltpu.VMEM),
            pl.BlockSpec(memory_space=pltpu.VMEM),
            pl.BlockSpec(memory_space=pltpu.VMEM),
            pl.BlockSpec(memory_space=pl.ANY),
            pl.BlockSpec(memory_space=pl.ANY),
        ],
        out_specs=pl.BlockSpec(memory_space=pltpu.VMEM),
        out_shape=jax.ShapeDtypeStruct((T, H), jnp.float32),
        scratch_shapes=[
            pltpu.VMEM((E, T), jnp.float32),
            pltpu.VMEM((1, S), jnp.int32),
            pltpu.VMEM((1, 1), jnp.int32),
            pltpu.SMEM((1, S), jnp.int32),
            pltpu.SMEM((1, 1), jnp.int32),
            pltpu.VMEM((NBUF, I2, H), jnp.float32),
            pltpu.VMEM((NBUF, H, I), jnp.float32),
            pltpu.SemaphoreType.DMA,
            pltpu.SemaphoreType.DMA,
            pltpu.SemaphoreType.DMA((NBUF,)),
            pltpu.SemaphoreType.DMA((NBUF,)),
        ],
        compiler_params=pltpu.CompilerParams(
            vmem_limit_bytes=100 * 1024 * 1024),
    )(mm, im, ss, iss, x, gate_up_proj, down_proj)


def kernel(x, router_logits, gate_up_proj, down_proj, top_k):
    del top_k
    return _fused_moe(x, router_logits, gate_up_proj, down_proj).astype(x.dtype)


# speedup vs baseline: 1.0025x; 1.0025x over previous
import functools

import jax
import jax.numpy as jnp
from jax import lax
from jax.experimental import pallas as pl
from jax.experimental.pallas import tpu as pltpu
from jax.experimental.pallas import tpu_sc as plsc

T, H, I2, E, K = 32, 768, 1536, 64, 2
I = I2 // 2
S = K * T
NBUF = 3
L = 16
NC = 2


def _sc_router_body(logits_hbm, val_out, idx_out, lv, fv, iv_):
    wid = lax.axis_index("s") * NC + lax.axis_index("c")
    pltpu.sync_copy(logits_hbm.at[wid], lv)

    iota = lax.iota(jnp.int32, L)
    m = lv[pl.ds(0, L)]
    im = iota
    s = jnp.full((L,), -jnp.inf, jnp.float32)
    is_ = jnp.zeros((L,), jnp.int32)
    for j in range(1, E // L):
        v = lv[pl.ds(L * j, L)]
        iv = L * j + iota
        ge = m >= v
        loser = jnp.where(ge, v, m)
        il = jnp.where(ge, iv, im)
        m = jnp.where(ge, m, v)
        im = jnp.where(ge, im, iv)
        ge2 = (loser > s) | ((loser == s) & (il < is_))
        s = jnp.where(ge2, loser, s)
        is_ = jnp.where(ge2, il, is_)

    fv[pl.ds(0, L)] = m
    fv[pl.ds(L, L)] = s
    iv_[pl.ds(0, L)] = im
    iv_[pl.ds(L, L)] = is_
    pltpu.sync_copy(fv, val_out.at[wid])
    pltpu.sync_copy(iv_, idx_out.at[wid])


_sc_router = functools.partial(
    pl.kernel,
    out_type=(
        jax.ShapeDtypeStruct((T, 2 * L), jnp.float32),
        jax.ShapeDtypeStruct((T, 2 * L), jnp.int32),
    ),
    mesh=plsc.VectorSubcoreMesh(core_axis_name="c", subcore_axis_name="s"),
    scratch_types=[
        pltpu.VMEM((E,), jnp.float32),
        pltpu.VMEM((2 * L,), jnp.float32),
        pltpu.VMEM((2 * L,), jnp.int32),
    ],
)(_sc_router_body)


def _row_of(col, n):
    i = jax.lax.broadcasted_iota(jnp.int32, (n, n), 0)
    j = jax.lax.broadcasted_iota(jnp.int32, (n, n), 1)
    b = jnp.broadcast_to(col, (n, n))
    return jnp.sum(jnp.where(i == j, b, jnp.zeros_like(b)), axis=0,
                   keepdims=True)


def _moe_body(val_ref, idx_ref, x_ref, gup_ref, dnp_ref, out_ref,
              wv, uv, nv, us, ns, gbuf, dbuf, ssem, nsem, gsem, dsem):
    vals = val_ref[...]
    idxs = idx_ref[...]
    m = vals[:, :L]
    im = idxs[:, :L]
    s = vals[:, L:]
    is_ = idxs[:, L:]
    m1v = jnp.max(m, axis=1, keepdims=True)
    i1 = jnp.min(jnp.where(m == m1v, im, E), axis=1, keepdims=True)
    chosen = im == i1
    cand = jnp.where(chosen, s, m)
    icand = jnp.where(chosen, is_, im)
    m2v = jnp.max(cand, axis=1, keepdims=True)
    i2 = jnp.min(jnp.where(cand == m2v, icand, E), axis=1, keepdims=True)
    e2 = jnp.exp(m2v - m1v)
    w1 = 1.0 / (1.0 + e2)
    w2 = 1.0 - w1

    e_col = jnp.concatenate([i1, i2], axis=0)
    e_row = _row_of(e_col, S)
    s_col = jax.lax.broadcasted_iota(jnp.int32, (S, S), 0)
    s_row = jax.lax.broadcasted_iota(jnp.int32, (S, S), 1)
    e_colb = jnp.broadcast_to(e_col, (S, S))
    e_rowb = jnp.broadcast_to(e_row, (S, S))
    same = e_rowb == e_colb
    dup_cnt = jnp.sum((same & (s_row < s_col)).astype(jnp.int32), axis=1,
                      keepdims=True)
    first = (dup_cnt == 0).astype(jnp.int32)
    firstb = jnp.broadcast_to(_row_of(first, S), (S, S))
    d = jnp.sum(((firstb == 1) & (e_rowb < e_colb)).astype(jnp.int32),
                axis=1, keepdims=True)
    j_row = jax.lax.broadcasted_iota(jnp.int32, (S, S), 1)
    put = (jnp.broadcast_to(d, (S, S)) == j_row) & (
        jnp.broadcast_to(first, (S, S)) == 1)
    uv[...] = jnp.sum(jnp.where(put, e_colb, jnp.zeros_like(e_colb)),
                      axis=0, keepdims=True)
    nv[...] = jnp.sum(first, keepdims=True)

    pltpu.make_async_copy(uv, us, ssem).start()
    pltpu.make_async_copy(nv, ns, nsem).start()
    pltpu.make_async_copy(uv, us, ssem).wait()
    pltpu.make_async_copy(nv, ns, nsem).wait()

    nu = ns[0, 0]

    def start_copy(u, slot):
        e = us[0, u]
        pltpu.make_async_copy(gup_ref.at[pl.ds(e, 1)],
                              gbuf.at[pl.ds(slot, 1)], gsem.at[slot]).start()
        pltpu.make_async_copy(dnp_ref.at[pl.ds(e, 1)],
                              dbuf.at[pl.ds(slot, 1)], dsem.at[slot]).start()

    for b in range(NBUF - 1):
        @pl.when(b < nu)
        def _pro():
            start_copy(b, b)

    i1r = jnp.broadcast_to(_row_of(i1, T), (E, T))
    i2r = jnp.broadcast_to(_row_of(i2, T), (E, T))
    w1r = jnp.broadcast_to(_row_of(w1, T), (E, T))
    w2r = jnp.broadcast_to(_row_of(w2, T), (E, T))
    e_iota = jax.lax.broadcasted_iota(jnp.int32, (E, T), 0)
    zero = jnp.zeros((E, T), jnp.float32)
    wv[...] = (jnp.where(e_iota == i1r, w1r, zero)
               + jnp.where(e_iota == i2r, w2r, zero))

    out_ref[...] = jnp.zeros_like(out_ref)

    def body(u, _):
        nxt = u + NBUF - 1

        @pl.when(nxt < nu)
        def _issue():
            start_copy(nxt, jax.lax.rem(nxt, NBUF))

        slot = jax.lax.rem(u, NBUF)
        e = us[0, u]
        pltpu.make_async_copy(gup_ref.at[pl.ds(e, 1)],
                              gbuf.at[pl.ds(slot, 1)], gsem.at[slot]).wait()
        pltpu.make_async_copy(dnp_ref.at[pl.ds(e, 1)],
                              dbuf.at[pl.ds(slot, 1)], dsem.at[slot]).wait()

        g = gbuf[slot]
        gu = jax.lax.dot_general(
            x_ref[...], g, (((1,), (1,)), ((), ())),
            preferred_element_type=jnp.float32)
        gate = gu[:, :I]
        up = gu[:, I:]
        act = gate * jax.nn.sigmoid(gate) * up
        dn = dbuf[slot]
        eo = jax.lax.dot_general(
            act, dn, (((1,), (1,)), ((), ())),
            preferred_element_type=jnp.float32)
        wrow = wv[pl.ds(e, 1), :]
        wcol = _col_of(wrow, T)
        out_ref[...] += wcol * eo
        return 0

    jax.lax.fori_loop(0, nu, body, 0)


def _col_of(row, n):
    i = jax.lax.broadcasted_iota(jnp.int32, (n, n), 0)
    j = jax.lax.broadcasted_iota(jnp.int32, (n, n), 1)
    b = jnp.broadcast_to(row, (n, n))
    return jnp.sum(jnp.where(i == j, b, jnp.zeros_like(b)), axis=1,
                   keepdims=True)


@jax.jit
def _fused_moe(x, router_logits, gate_up_proj, down_proj):
    scvals, scidxs = _sc_router(router_logits)
    return pl.pallas_call(
        _moe_body,
        in_specs=[
            pl.BlockSpec(memory_space=pltpu.VMEM),
            pl.BlockSpec(memory_space=pltpu.VMEM),
            pl.BlockSpec(memory_space=pltpu.VMEM),
            pl.BlockSpec(memory_space=pl.ANY),
            pl.BlockSpec(memory_space=pl.ANY),
        ],
        out_specs=pl.BlockSpec(memory_space=pltpu.VMEM),
        out_shape=jax.ShapeDtypeStruct((T, H), jnp.float32),
        scratch_shapes=[
            pltpu.VMEM((E, T), jnp.float32),
            pltpu.VMEM((1, S), jnp.int32),
            pltpu.VMEM((1, 1), jnp.int32),
            pltpu.SMEM((1, S), jnp.int32),
            pltpu.SMEM((1, 1), jnp.int32),
            pltpu.VMEM((NBUF, I2, H), jnp.float32),
            pltpu.VMEM((NBUF, H, I), jnp.float32),
            pltpu.SemaphoreType.DMA,
            pltpu.SemaphoreType.DMA,
            pltpu.SemaphoreType.DMA((NBUF,)),
            pltpu.SemaphoreType.DMA((NBUF,)),
        ],
        compiler_params=pltpu.CompilerParams(
            vmem_limit_bytes=100 * 1024 * 1024),
    )(scvals, scidxs, x, gate_up_proj, down_proj)


def kernel(x, router_logits, gate_up_proj, down_proj, top_k):
    del top_k
    return _fused_moe(x, router_logits, gate_up_proj, down_proj).astype(x.dtype)
